# Initial kernel scaffold; baseline (speedup 1.0000x reference)
import jax
import jax.numpy as jnp
from jax.experimental import pallas as pl


def kernel(X):
    raise NotImplementedError("write your pallas kernel here")


# speedup vs baseline: 7.3697x; 7.3697x over previous
import jax
import jax.numpy as jnp
from jax.experimental import pallas as pl
from jax.experimental.pallas import tpu as pltpu

N = 8192
D = 512
BLK = 1024
NB = N // BLK
LAMBDA = 0.5


def _body(x_ref, out_ref, acc_ref):
    phase = pl.program_id(0)
    i = pl.program_id(1)

    x = x_ref[...]
    q = jnp.sum(x * x, axis=1, keepdims=True)
    rinv = jax.lax.rsqrt(q)

    @pl.when(jnp.logical_and(phase == 0, i == 0))
    def _init():
        acc_ref[...] = jnp.zeros_like(acc_ref)

    @pl.when(phase == 0)
    def _accumulate():
        acc_ref[...] += jnp.sum(x * rinv, axis=0, keepdims=True)

    @pl.when(phase == 1)
    def _gains():
        s = acc_ref[...]
        p = jnp.sum(x * s, axis=1, keepdims=True)
        g = p * rinv - LAMBDA * q * rinv * rinv
        out_ref[...] = g.T


def kernel(X):
    out = pl.pallas_call(
        _body,
        grid=(2, NB),
        in_specs=[pl.BlockSpec((BLK, D), lambda p, i: (i, 0))],
        out_specs=pl.BlockSpec((1, BLK), lambda p, i: (0, i)),
        out_shape=jax.ShapeDtypeStruct((1, N), jnp.float32),
        scratch_shapes=[pltpu.VMEM((1, D), jnp.float32)],
    )(X)
    return out.reshape(N)


# speedup vs baseline: 9.4047x; 1.2761x over previous
import jax
import jax.numpy as jnp
from jax.experimental import pallas as pl
from jax.experimental.pallas import tpu as pltpu

N = 8192
D = 512
BLK = 1024
NB = N // BLK
LAMBDA = 0.5


def _body(x_ref, out_ref, xn_ref, acc_ref):
    phase = pl.program_id(0)
    i = pl.program_id(1)

    @pl.when(jnp.logical_and(phase == 0, i == 0))
    def _init():
        acc_ref[...] = jnp.zeros_like(acc_ref)

    @pl.when(phase == 0)
    def _accumulate():
        x = x_ref[...]
        q = jnp.sum(x * x, axis=1, keepdims=True)
        xn = x * jax.lax.rsqrt(q)
        xn_ref[pl.ds(i * BLK, BLK), :] = xn
        acc_ref[...] += jnp.sum(xn, axis=0, keepdims=True)

    @pl.when(phase == 1)
    def _gains():
        xn = xn_ref[pl.ds(i * BLK, BLK), :]
        s = acc_ref[...]
        p = jnp.sum(xn * s, axis=1, keepdims=True)
        d = jnp.sum(xn * xn, axis=1, keepdims=True)
        out_ref[...] = (p - LAMBDA * d).T


def kernel(X):
    out = pl.pallas_call(
        _body,
        grid=(2, NB),
        in_specs=[pl.BlockSpec((BLK, D), lambda p, i: (i * (1 - p), 0))],
        out_specs=pl.BlockSpec((1, BLK), lambda p, i: (0, i)),
        out_shape=jax.ShapeDtypeStruct((1, N), jnp.float32),
        scratch_shapes=[
            pltpu.VMEM((N, D), jnp.float32),
            pltpu.VMEM((1, D), jnp.float32),
        ],
    )(X)
    return out.reshape(N)
